# baseline (device time: 231273 ns/iter reference)
import jax
import jax.numpy as jnp
from jax import lax
from jax.experimental import pallas as pl
from jax.experimental.pallas import tpu as pltpu

N_DEV = 4
HQ = 8
DH = 128
SQ = 1024
SKV = 1024
D_MODEL = 1024
GD = HQ * DH
SCALE = 0.08838834764831843

_PERM = [(4 * u + t, 4 * t + u) for u in range(4) for t in range(4)]


def _attend(q_all, k_ref, v_ref, ctx_ref):
    for h in range(HQ):
        cs = slice(h * DH, (h + 1) * DH)
        q4 = q_all[:, cs].reshape(4, 256, DH)
        k4 = k_ref[:, cs].reshape(4, 256, DH)
        v4 = v_ref[:, cs].reshape(4, 256, DH)
        scores = lax.dot_general(
            q4, k4, (((2,), (2,)), ((0,), (0,))),
            preferred_element_type=jnp.float32) * SCALE
        e = jnp.exp(scores)
        z = jnp.sum(e, axis=2, keepdims=True)
        w = (e / z).astype(jnp.bfloat16)
        c = lax.dot_general(
            w, v4, (((2,), (1,)), ((0,), (0,))),
            preferred_element_type=jnp.float32)
        ctx_ref[:, cs] = c.astype(jnp.bfloat16).reshape(SQ, DH)


def _body(x_hbm, wq_ref, wo_ref, k_hbm, v_hbm, out_ref,
          comm, kbf, vbf, stage, xp, ctx_ref, acc,
          ssem, rsem, dma_sem):
    my = lax.axis_index("i")
    left = lax.rem(my + N_DEV - 1, N_DEV)
    right = lax.rem(my + 1, N_DEV)

    def stage_dma(src3, rows_pref, col_off, t):
        c0 = pltpu.make_async_copy(
            src3.at[rows_pref, pl.ds(0, 512), pl.ds(col_off, GD)],
            stage.at[t, 0], dma_sem.at[t, 0])
        c1 = pltpu.make_async_copy(
            src3.at[rows_pref, pl.ds(512, 512), pl.ds(col_off, GD)],
            stage.at[t, 1], dma_sem.at[t, 1])
        c0.start()
        c1.start()
        return c0, c1

    def convert_perm(t, dst_ref):
        for d, s in _PERM:
            half, row = divmod(s, 8)
            dst_ref[pl.ds(64 * d, 64), :] = stage[
                t, half, pl.ds(64 * row, 64), :].astype(jnp.bfloat16)

    gorder = [my, left, right, lax.rem(my + 2, N_DEV)]

    cx = stage_dma(x_hbm, 0, 0, 0)
    ck = stage_dma(k_hbm, my, gorder[0] * GD, 1)

    barrier = pltpu.get_barrier_semaphore()
    pl.semaphore_signal(barrier, inc=1, device_id=(left,),
                        device_id_type=pl.DeviceIdType.MESH)
    pl.semaphore_signal(barrier, inc=1, device_id=(right,),
                        device_id_type=pl.DeviceIdType.MESH)
    pl.semaphore_wait(barrier, 2)

    comm[0, :D_MODEL, :] = wq_ref[...].astype(jnp.bfloat16)

    def rdma(src_slot, src_half, dst_slot, sem_i, dev):
        return pltpu.make_async_remote_copy(
            src_ref=comm.at[src_slot, pl.ds(src_half * D_MODEL, D_MODEL)],
            dst_ref=comm.at[dst_slot, pl.ds(src_half * D_MODEL, D_MODEL)],
            send_sem=ssem.at[sem_i], recv_sem=rsem.at[sem_i],
            device_id=(dev,), device_id_type=pl.DeviceIdType.MESH)

    r1r_wq = rdma(0, 0, 1, 0, right)
    r1l_wq = rdma(0, 0, 2, 2, left)
    r1r_wq.start()
    r1l_wq.start()
    comm[0, D_MODEL:, :] = wo_ref[...].astype(jnp.bfloat16)
    r1r_wo = rdma(0, 1, 1, 1, right)
    r1l_wo = rdma(0, 1, 2, 3, left)
    r1r_wo.start()
    r1l_wo.start()
    r2r = rdma(1, 0, 3, 4, right)
    r2l = rdma(2, 1, 3, 5, left)

    cx[0].wait()
    cx[1].wait()
    convert_perm(0, xp)
    cv = stage_dma(v_hbm, my, gorder[0] * GD, 0)
    ck[0].wait()
    ck[1].wait()
    convert_perm(1, kbf.at[0])
    cv[0].wait()
    cv[1].wait()
    convert_perm(0, vbf.at[0])

    x = xp[...]

    for s in range(4):
        if s < 3:
            cks = stage_dma(k_hbm, my, gorder[s + 1] * GD, 1)
            cvs = stage_dma(v_hbm, my, gorder[s + 1] * GD, 0)
        if s == 1:
            r1r_wq.wait_recv()
            r2r.start()
        if s == 2:
            r1l_wq.wait_recv()
        if s == 3:
            r2r.wait_recv()

        q_all = jnp.dot(x, comm[s, :D_MODEL, :],
                        preferred_element_type=jnp.float32).astype(jnp.bfloat16)
        _attend(q_all, kbf.at[s % 2], vbf.at[s % 2], ctx_ref)

        if s == 1:
            r1r_wo.wait_recv()
        if s == 2:
            r1l_wo.wait_recv()
            r2l.start()
        if s == 3:
            r2l.wait_recv()
        proj = jnp.dot(ctx_ref[...], comm[s, D_MODEL:, :],
                       preferred_element_type=jnp.float32)
        if s == 0:
            acc[...] = proj
        else:
            acc[...] += proj

        if s < 3:
            cks[0].wait()
            cks[1].wait()
            convert_perm(1, kbf.at[(s + 1) % 2])
            cvs[0].wait()
            cvs[1].wait()
            convert_perm(0, vbf.at[(s + 1) % 2])

    for d, s in _PERM:
        out_ref[0, pl.ds(64 * d, 64), :] = acc[pl.ds(64 * s, 64), :]

    for r in (r1r_wq, r1l_wq, r1r_wo, r1l_wo, r2r, r2l):
        r.wait_send()


def kernel(x, Wq, K_ext, V_ext, Wo):
    kb = K_ext.reshape(N_DEV, SKV, N_DEV * GD)
    vb = V_ext.reshape(N_DEV, SKV, N_DEV * GD)

    return pl.pallas_call(
        _body,
        out_shape=jax.ShapeDtypeStruct((1, SQ, D_MODEL), jnp.float32),
        in_specs=[
            pl.BlockSpec(memory_space=pltpu.MemorySpace.HBM),
            pl.BlockSpec(memory_space=pltpu.VMEM),
            pl.BlockSpec(memory_space=pltpu.VMEM),
            pl.BlockSpec(memory_space=pltpu.MemorySpace.HBM),
            pl.BlockSpec(memory_space=pltpu.MemorySpace.HBM),
        ],
        out_specs=pl.BlockSpec(memory_space=pltpu.VMEM),
        scratch_shapes=[
            pltpu.VMEM((4, 2 * D_MODEL, GD), jnp.bfloat16),
            pltpu.VMEM((2, SKV, GD), jnp.bfloat16),
            pltpu.VMEM((2, SKV, GD), jnp.bfloat16),
            pltpu.VMEM((2, 2, 512, GD), jnp.float32),
            pltpu.VMEM((SQ, D_MODEL), jnp.bfloat16),
            pltpu.VMEM((SQ, GD), jnp.bfloat16),
            pltpu.VMEM((SQ, D_MODEL), jnp.float32),
            pltpu.SemaphoreType.DMA((6,)),
            pltpu.SemaphoreType.DMA((6,)),
            pltpu.SemaphoreType.DMA((2, 2)),
        ],
        compiler_params=pltpu.CompilerParams(
            collective_id=0, vmem_limit_bytes=100 * 1024 * 1024),
    )(x, Wq, Wo, kb, vb)


# device time: 169001 ns/iter; 1.3685x vs baseline; 1.3685x over previous
import jax
import jax.numpy as jnp
from jax import lax
from jax.experimental import pallas as pl
from jax.experimental.pallas import tpu as pltpu

N_DEV = 4
HQ = 8
DH = 128
SQ = 1024
SKV = 1024
D_MODEL = 1024
GD = HQ * DH
SCALE = 0.08838834764831843

_PERM = [(4 * u + t, 4 * t + u) for u in range(4) for t in range(4)]


def _attend(q_all, k_ref, v_ref, ctx_ref):
    for h in range(HQ):
        cs = slice(h * DH, (h + 1) * DH)
        q4 = q_all[:, cs].reshape(4, 256, DH)
        k4 = k_ref[:, cs].reshape(4, 256, DH)
        v4 = v_ref[:, cs].reshape(4, 256, DH)
        scores = lax.dot_general(
            q4, k4, (((2,), (2,)), ((0,), (0,))),
            preferred_element_type=jnp.float32) * SCALE
        e = jnp.exp(scores)
        z = jnp.sum(e, axis=2, keepdims=True)
        w = (e / z).astype(jnp.bfloat16)
        c = lax.dot_general(
            w, v4, (((2,), (1,)), ((0,), (0,))),
            preferred_element_type=jnp.float32)
        ctx_ref[:, cs] = c.astype(jnp.bfloat16).reshape(SQ, DH)


def _body(x_ref, wq_ref, wo_ref, k_ref, v_ref, out_ref,
          comm, ctx_ref, acc, ssem, rsem):
    my = lax.axis_index("i")
    left = lax.rem(my + N_DEV - 1, N_DEV)
    right = lax.rem(my + 1, N_DEV)

    barrier = pltpu.get_barrier_semaphore()
    pl.semaphore_signal(barrier, inc=1, device_id=(left,),
                        device_id_type=pl.DeviceIdType.MESH)
    pl.semaphore_signal(barrier, inc=1, device_id=(right,),
                        device_id_type=pl.DeviceIdType.MESH)
    pl.semaphore_wait(barrier, 2)

    comm[0, :D_MODEL, :] = wq_ref[...]

    def rdma(src_slot, half, dst_slot, sem_i, dev):
        return pltpu.make_async_remote_copy(
            src_ref=comm.at[src_slot, pl.ds(half * D_MODEL, D_MODEL)],
            dst_ref=comm.at[dst_slot, pl.ds(half * D_MODEL, D_MODEL)],
            send_sem=ssem.at[sem_i], recv_sem=rsem.at[sem_i],
            device_id=(dev,), device_id_type=pl.DeviceIdType.MESH)

    r1r_wq = rdma(0, 0, 1, 0, right)
    r1l_wq = rdma(0, 0, 2, 2, left)
    r1r_wq.start()
    r1l_wq.start()
    comm[0, D_MODEL:, :] = wo_ref[...]
    r1r_wo = rdma(0, 1, 1, 1, right)
    r1l_wo = rdma(0, 1, 2, 3, left)
    r1r_wo.start()
    r1l_wo.start()
    r2r = rdma(1, 0, 3, 4, right)
    r2l = rdma(2, 1, 3, 5, left)

    x = x_ref[...]

    gorder = [my, left, right, lax.rem(my + 2, N_DEV)]

    for s in range(4):
        if s == 1:
            r1r_wq.wait_recv()
            r2r.start()
        if s == 2:
            r1l_wq.wait_recv()
        if s == 3:
            r2r.wait_recv()
        g = gorder[s]

        q_all = jnp.dot(x, comm[s, :D_MODEL, :],
                        preferred_element_type=jnp.float32).astype(jnp.bfloat16)
        _attend(q_all, k_ref.at[g], v_ref.at[g], ctx_ref)

        if s == 1:
            r1r_wo.wait_recv()
        if s == 2:
            r1l_wo.wait_recv()
            r2l.start()
        if s == 3:
            r2l.wait_recv()
        proj = jnp.dot(ctx_ref[...], comm[s, D_MODEL:, :],
                       preferred_element_type=jnp.float32)
        if s == 0:
            acc[...] = proj
        else:
            acc[...] += proj

    for d, s in _PERM:
        out_ref[0, pl.ds(64 * d, 64), :] = acc[pl.ds(64 * s, 64), :]

    for r in (r1r_wq, r1l_wq, r1r_wo, r1l_wo, r2r, r2l):
        r.wait_send()


def _permute_rows(a):
    return a.reshape(4, 4, 64, *a.shape[1:]).swapaxes(0, 1).reshape(a.shape)


def kernel(x, Wq, K_ext, V_ext, Wo):
    my = lax.axis_index("i")

    xb = _permute_rows(x[0].astype(jnp.bfloat16))
    wq = Wq.astype(jnp.bfloat16)
    wo = Wo.astype(jnp.bfloat16)

    kb = lax.dynamic_index_in_dim(K_ext, my, 0, keepdims=False)
    vb = lax.dynamic_index_in_dim(V_ext, my, 0, keepdims=False)
    kb = _permute_rows(kb.astype(jnp.bfloat16).reshape(SKV, 4 * GD))
    vb = _permute_rows(vb.astype(jnp.bfloat16).reshape(SKV, 4 * GD))
    kb = kb.reshape(SKV, N_DEV, GD).transpose(1, 0, 2)
    vb = vb.reshape(SKV, N_DEV, GD).transpose(1, 0, 2)

    return pl.pallas_call(
        _body,
        out_shape=jax.ShapeDtypeStruct((1, SQ, D_MODEL), jnp.float32),
        in_specs=[pl.BlockSpec(memory_space=pltpu.VMEM)] * 5,
        out_specs=pl.BlockSpec(memory_space=pltpu.VMEM),
        scratch_shapes=[
            pltpu.VMEM((4, 2 * D_MODEL, GD), jnp.bfloat16),
            pltpu.VMEM((SQ, GD), jnp.bfloat16),
            pltpu.VMEM((SQ, D_MODEL), jnp.float32),
            pltpu.SemaphoreType.DMA((6,)),
            pltpu.SemaphoreType.DMA((6,)),
        ],
        compiler_params=pltpu.CompilerParams(
            collective_id=0, vmem_limit_bytes=100 * 1024 * 1024),
    )(xb, wq, wo, kb, vb)


# device time: 168833 ns/iter; 1.3698x vs baseline; 1.0010x over previous
import jax
import jax.numpy as jnp
from jax import lax
from jax.experimental import pallas as pl
from jax.experimental.pallas import tpu as pltpu

N_DEV = 4
HQ = 8
DH = 128
SQ = 1024
SKV = 1024
D_MODEL = 1024
GD = HQ * DH
SCALE = 0.08838834764831843

_PERM = [(4 * u + t, 4 * t + u) for u in range(4) for t in range(4)]


def _attend(q_all, k_all, v_all, ctx_ref):
    for h in range(HQ):
        cs = slice(h * DH, (h + 1) * DH)
        q4 = q_all[:, cs].reshape(4, 256, DH)
        k4 = k_all[:, cs].reshape(4, 256, DH)
        v4 = v_all[:, cs].reshape(4, 256, DH)
        scores = lax.dot_general(
            q4, k4, (((2,), (2,)), ((0,), (0,))),
            preferred_element_type=jnp.float32) * SCALE
        e = jnp.exp(scores)
        z = jnp.sum(e, axis=2, keepdims=True)
        w = (e / z).astype(jnp.bfloat16)
        c = lax.dot_general(
            w, v4, (((2,), (1,)), ((0,), (0,))),
            preferred_element_type=jnp.float32)
        ctx_ref[:, cs] = c.astype(jnp.bfloat16).reshape(SQ, DH)


def _body(x_ref, wq_ref, wo_ref, k_ref, v_ref, out_ref,
          comm, ctx_ref, acc, ssem, rsem):
    my = lax.axis_index("i")
    left = lax.rem(my + N_DEV - 1, N_DEV)
    right = lax.rem(my + 1, N_DEV)

    barrier = pltpu.get_barrier_semaphore()
    pl.semaphore_signal(barrier, inc=1, device_id=(left,),
                        device_id_type=pl.DeviceIdType.MESH)
    pl.semaphore_signal(barrier, inc=1, device_id=(right,),
                        device_id_type=pl.DeviceIdType.MESH)
    pl.semaphore_wait(barrier, 2)

    comm[0, :D_MODEL, :] = wq_ref[...]

    def rdma(src_slot, half, dst_slot, sem_i, dev):
        return pltpu.make_async_remote_copy(
            src_ref=comm.at[src_slot, pl.ds(half * D_MODEL, D_MODEL)],
            dst_ref=comm.at[dst_slot, pl.ds(half * D_MODEL, D_MODEL)],
            send_sem=ssem.at[sem_i], recv_sem=rsem.at[sem_i],
            device_id=(dev,), device_id_type=pl.DeviceIdType.MESH)

    r1r_wq = rdma(0, 0, 1, 0, right)
    r1l_wq = rdma(0, 0, 2, 2, left)
    r1r_wq.start()
    r1l_wq.start()
    comm[0, D_MODEL:, :] = wo_ref[...]
    r1r_wo = rdma(0, 1, 1, 1, right)
    r1l_wo = rdma(0, 1, 2, 3, left)
    r1r_wo.start()
    r1l_wo.start()
    r2r = rdma(1, 0, 3, 4, right)
    r2l = rdma(2, 1, 3, 5, left)

    x = x_ref[...]

    gorder = [my, left, right, lax.rem(my + 2, N_DEV)]

    for s in range(4):
        if s == 1:
            r1r_wq.wait_recv()
            r2r.start()
        if s == 2:
            r1l_wq.wait_recv()
        if s == 3:
            r2r.wait_recv()
        g = gorder[s]

        q_all = jnp.dot(x, comm[s, :D_MODEL, :],
                        preferred_element_type=jnp.float32).astype(jnp.bfloat16)
        _attend(q_all, k_ref[g], v_ref[g], ctx_ref)

        if s == 1:
            r1r_wo.wait_recv()
        if s == 2:
            r1l_wo.wait_recv()
            r2l.start()
        if s == 3:
            r2l.wait_recv()
        proj = jnp.dot(ctx_ref[...], comm[s, D_MODEL:, :],
                       preferred_element_type=jnp.float32)
        if s == 0:
            acc[...] = proj
        else:
            acc[...] += proj

    for d, s in _PERM:
        out_ref[0, pl.ds(64 * d, 64), :] = acc[pl.ds(64 * s, 64), :]

    for r in (r1r_wq, r1l_wq, r1r_wo, r1l_wo, r2r, r2l):
        r.wait_send()


def _permute_rows(a):
    return a.reshape(4, 4, 64, *a.shape[1:]).swapaxes(0, 1).reshape(a.shape)


def kernel(x, Wq, K_ext, V_ext, Wo):
    my = lax.axis_index("i")

    xb = _permute_rows(x[0].astype(jnp.bfloat16))
    wq = Wq.astype(jnp.bfloat16)
    wo = Wo.astype(jnp.bfloat16)

    kb = lax.dynamic_index_in_dim(K_ext, my, 0, keepdims=False)
    vb = lax.dynamic_index_in_dim(V_ext, my, 0, keepdims=False)
    kb = _permute_rows(kb.astype(jnp.bfloat16).reshape(SKV, 4 * GD))
    vb = _permute_rows(vb.astype(jnp.bfloat16).reshape(SKV, 4 * GD))
    kb = kb.reshape(SKV, N_DEV, GD).transpose(1, 0, 2)
    vb = vb.reshape(SKV, N_DEV, GD).transpose(1, 0, 2)

    return pl.pallas_call(
        _body,
        out_shape=jax.ShapeDtypeStruct((1, SQ, D_MODEL), jnp.float32),
        in_specs=[pl.BlockSpec(memory_space=pltpu.VMEM)] * 5,
        out_specs=pl.BlockSpec(memory_space=pltpu.VMEM),
        scratch_shapes=[
            pltpu.VMEM((4, 2 * D_MODEL, GD), jnp.bfloat16),
            pltpu.VMEM((SQ, GD), jnp.bfloat16),
            pltpu.VMEM((SQ, D_MODEL), jnp.float32),
            pltpu.SemaphoreType.DMA((6,)),
            pltpu.SemaphoreType.DMA((6,)),
        ],
        compiler_params=pltpu.CompilerParams(
            collective_id=0, vmem_limit_bytes=100 * 1024 * 1024),
    )(xb, wq, wo, kb, vb)


# device time: 165016 ns/iter; 1.4015x vs baseline; 1.0231x over previous
import jax
import jax.numpy as jnp
from jax import lax
from jax.experimental import pallas as pl
from jax.experimental.pallas import tpu as pltpu

N_DEV = 4
HQ = 8
DH = 128
SQ = 1024
SKV = 1024
D_MODEL = 1024
GD = HQ * DH
SCALE = 0.08838834764831843

_PERM = [(4 * u + t, 4 * t + u) for u in range(4) for t in range(4)]


def _attend(q_all, k_all, v_all, ctx_ref):
    for h in range(HQ):
        cs = slice(h * DH, (h + 1) * DH)
        q4 = q_all[:, cs].reshape(4, 256, DH)
        k4 = k_all[:, cs].reshape(4, 256, DH)
        v4 = v_all[:, cs].reshape(4, 256, DH)
        scores = lax.dot_general(
            q4, k4, (((2,), (2,)), ((0,), (0,))),
            preferred_element_type=jnp.float32) * SCALE
        e = jnp.exp(scores)
        z = jnp.sum(e, axis=2, keepdims=True)
        w = (e / z).astype(jnp.bfloat16)
        c = lax.dot_general(
            w, v4, (((2,), (1,)), ((0,), (0,))),
            preferred_element_type=jnp.float32)
        ctx_ref[:, cs] = c.astype(jnp.bfloat16).reshape(SQ, DH)


def _body(x_ref, wq_ref, wo_ref, k_ref, v_ref, out_ref,
          comm, ctx_ref, acc, xp, ssem, rsem):
    my = lax.axis_index("i")
    left = lax.rem(my + N_DEV - 1, N_DEV)
    right = lax.rem(my + 1, N_DEV)

    barrier = pltpu.get_barrier_semaphore()
    pl.semaphore_signal(barrier, inc=1, device_id=(left,),
                        device_id_type=pl.DeviceIdType.MESH)
    pl.semaphore_signal(barrier, inc=1, device_id=(right,),
                        device_id_type=pl.DeviceIdType.MESH)
    pl.semaphore_wait(barrier, 2)

    comm[0, :D_MODEL, :] = wq_ref[...]

    def rdma(src_slot, half, dst_slot, sem_i, dev):
        return pltpu.make_async_remote_copy(
            src_ref=comm.at[src_slot, pl.ds(half * D_MODEL, D_MODEL)],
            dst_ref=comm.at[dst_slot, pl.ds(half * D_MODEL, D_MODEL)],
            send_sem=ssem.at[sem_i], recv_sem=rsem.at[sem_i],
            device_id=(dev,), device_id_type=pl.DeviceIdType.MESH)

    r1r_wq = rdma(0, 0, 1, 0, right)
    r1l_wq = rdma(0, 0, 2, 2, left)
    r1r_wq.start()
    r1l_wq.start()
    comm[0, D_MODEL:, :] = wo_ref[...]
    r1r_wo = rdma(0, 1, 1, 1, right)
    r1l_wo = rdma(0, 1, 2, 3, left)
    r1r_wo.start()
    r1l_wo.start()
    r2r = rdma(1, 0, 3, 4, right)
    r2l = rdma(2, 1, 3, 5, left)

    for d, s in _PERM:
        xp[pl.ds(64 * d, 64), :] = x_ref[0, pl.ds(64 * s, 64), :].astype(
            jnp.bfloat16)
    x = xp[...]

    gorder = [my, left, right, lax.rem(my + 2, N_DEV)]

    for s in range(4):
        if s == 1:
            r1r_wq.wait_recv()
            r2r.start()
        if s == 2:
            r1l_wq.wait_recv()
        if s == 3:
            r2r.wait_recv()
        g = gorder[s]

        q_all = jnp.dot(x, comm[s, :D_MODEL, :],
                        preferred_element_type=jnp.float32).astype(jnp.bfloat16)
        _attend(q_all, k_ref[g], v_ref[g], ctx_ref)

        if s == 1:
            r1r_wo.wait_recv()
        if s == 2:
            r1l_wo.wait_recv()
            r2l.start()
        if s == 3:
            r2l.wait_recv()
        proj = jnp.dot(ctx_ref[...], comm[s, D_MODEL:, :],
                       preferred_element_type=jnp.float32)
        if s == 0:
            acc[...] = proj
        else:
            acc[...] += proj

    for d, s in _PERM:
        out_ref[0, pl.ds(64 * d, 64), :] = acc[
            pl.ds(64 * s, 64), :].astype(jnp.bfloat16)

    for r in (r1r_wq, r1l_wq, r1r_wo, r1l_wo, r2r, r2l):
        r.wait_send()


def _permute_rows(a):
    return a.reshape(4, 4, 64, *a.shape[1:]).swapaxes(0, 1).reshape(a.shape)


def kernel(x, Wq, K_ext, V_ext, Wo):
    my = lax.axis_index("i")

    wq = Wq.astype(jnp.bfloat16)
    wo = Wo.astype(jnp.bfloat16)

    kb = lax.dynamic_index_in_dim(K_ext, my, 0, keepdims=False)
    vb = lax.dynamic_index_in_dim(V_ext, my, 0, keepdims=False)
    kb = _permute_rows(kb.astype(jnp.bfloat16).reshape(SKV, 4 * GD))
    vb = _permute_rows(vb.astype(jnp.bfloat16).reshape(SKV, 4 * GD))
    kb = kb.reshape(SKV, N_DEV, GD).transpose(1, 0, 2)
    vb = vb.reshape(SKV, N_DEV, GD).transpose(1, 0, 2)

    return pl.pallas_call(
        _body,
        out_shape=jax.ShapeDtypeStruct((1, SQ, D_MODEL), jnp.bfloat16),
        in_specs=[pl.BlockSpec(memory_space=pltpu.VMEM)] * 5,
        out_specs=pl.BlockSpec(memory_space=pltpu.VMEM),
        scratch_shapes=[
            pltpu.VMEM((4, 2 * D_MODEL, GD), jnp.bfloat16),
            pltpu.VMEM((SQ, GD), jnp.bfloat16),
            pltpu.VMEM((SQ, D_MODEL), jnp.float32),
            pltpu.VMEM((SQ, D_MODEL), jnp.bfloat16),
            pltpu.SemaphoreType.DMA((6,)),
            pltpu.SemaphoreType.DMA((6,)),
        ],
        compiler_params=pltpu.CompilerParams(
            collective_id=0, vmem_limit_bytes=100 * 1024 * 1024),
    )(x, wq, wo, kb, vb)
